# baseline (speedup 1.0000x reference)
import functools

import jax
import jax.numpy as jnp
from jax import lax
from jax.experimental import pallas as pl
from jax.experimental.pallas import tpu as pltpu
from jax.experimental.pallas import tpu_sc as plsc

NUM_UNIQUE = 1000000
EMBED_DIM = 64
BATCH = 16384

_NC = 2
_NS = 16
_NW = _NC * _NS
_B_PER_W = BATCH // _NW
_CHUNK = 128
_NCHUNK = _B_PER_W // _CHUNK


def _make_gather():
    mesh = plsc.VectorSubcoreMesh(core_axis_name="c", subcore_axis_name="s")

    @functools.partial(
        pl.kernel,
        mesh=mesh,
        out_type=jax.ShapeDtypeStruct((BATCH, EMBED_DIM), jnp.float32),
        scratch_types=[
            pltpu.VMEM((_NCHUNK, _CHUNK), jnp.int32),
            pltpu.VMEM((_B_PER_W, EMBED_DIM), jnp.float32),
            pltpu.SemaphoreType.DMA,
        ],
        compiler_params=pltpu.CompilerParams(use_tc_tiling_on_sc=False),
    )
    def k(table_hbm, idx_hbm, out_hbm, idx_v, rows_v, sem):
        wid = lax.axis_index("s") * _NC + lax.axis_index("c")
        pltpu.sync_copy(idx_hbm.at[pl.ds(wid * _NCHUNK, _NCHUNK)], idx_v)
        copies = []
        for c in range(_NCHUNK):
            copies.append(
                pltpu.async_copy(
                    table_hbm.at[idx_v.at[c]],
                    rows_v.at[pl.ds(c * _CHUNK, _CHUNK)],
                    sem,
                )
            )
        for cp in copies:
            cp.wait()
        pltpu.sync_copy(rows_v, out_hbm.at[pl.ds(wid * _B_PER_W, _B_PER_W)])

    return k


_gather = _make_gather()


def kernel(x, table):
    idx2d = x.astype(jnp.int32).reshape(BATCH // _CHUNK, _CHUNK)
    return _gather(table, idx2d)


# speedup vs baseline: 1.6891x; 1.6891x over previous
import functools

import jax
import jax.numpy as jnp
from jax import lax
from jax.experimental import pallas as pl
from jax.experimental.pallas import tpu as pltpu
from jax.experimental.pallas import tpu_sc as plsc

NUM_UNIQUE = 1000000
EMBED_DIM = 64
BATCH = 16384

_NC = 2
_NS = 16
_NW = _NC * _NS
_B_PER_W = BATCH // _NW
_K = 8


def _make_gather():
    mesh = plsc.VectorSubcoreMesh(core_axis_name="c", subcore_axis_name="s")

    @functools.partial(
        pl.kernel,
        mesh=mesh,
        out_type=jax.ShapeDtypeStruct((BATCH, EMBED_DIM), jnp.float32),
        scratch_types=[
            pltpu.VMEM((4, 128), jnp.int32),
            pltpu.VMEM((_B_PER_W // 8, 8, EMBED_DIM), jnp.float32),
            pltpu.SemaphoreType.DMA,
        ],
    )
    def k(table_hbm, idx_hbm, out_hbm, idx_v, rows_v, sem):
        wid = lax.axis_index("s") * _NC + lax.axis_index("c")
        pltpu.sync_copy(idx_hbm.at[wid], idx_v)

        def drain_one():
            pltpu.make_async_copy(table_hbm.at[0], rows_v.at[0, 0], sem).wait()

        def body(g, carry):
            tv = idx_v[g >> 3, pl.ds((g & 7) * 16, 16)]
            for l in range(16):
                pltpu.async_copy(
                    table_hbm.at[tv[l]],
                    rows_v.at[2 * g + (l // 8), l & 7],
                    sem,
                )

            @pl.when(g >= 1)
            def _():
                for _ in range(16):
                    drain_one()

            return carry

        lax.fori_loop(0, _B_PER_W // 16, body, 0)
        for _ in range(16):
            drain_one()

        out3 = out_hbm.reshape(BATCH // 8, 8, EMBED_DIM)
        pltpu.sync_copy(rows_v, out3.at[pl.ds(wid * (_B_PER_W // 8),
                                              _B_PER_W // 8)])

    return k


_gather = _make_gather()


def kernel(x, table):
    idx3 = x.astype(jnp.int32).reshape(_NW, 4, 128)
    return _gather(table, idx3)


# speedup vs baseline: 1.7267x; 1.0222x over previous
import functools

import jax
import jax.numpy as jnp
from jax import lax
from jax.experimental import pallas as pl
from jax.experimental.pallas import tpu as pltpu
from jax.experimental.pallas import tpu_sc as plsc

NUM_UNIQUE = 1000000
EMBED_DIM = 64
BATCH = 16384

_NC = 2
_NS = 16
_NW = _NC * _NS
_B_PER_W = BATCH // _NW
_CHUNK = 128
_NCHUNK = _B_PER_W // _CHUNK


def _make_gather():
    mesh = plsc.VectorSubcoreMesh(core_axis_name="c", subcore_axis_name="s")

    @functools.partial(
        pl.kernel,
        mesh=mesh,
        out_type=jax.ShapeDtypeStruct((BATCH, EMBED_DIM), jnp.float32),
        scratch_types=[
            pltpu.VMEM((4, 128), jnp.int32),
            pltpu.VMEM((_B_PER_W // 8, 8, EMBED_DIM), jnp.float32),
            pltpu.SemaphoreType.DMA,
            pltpu.SemaphoreType.DMA,
            pltpu.SemaphoreType.DMA,
            pltpu.SemaphoreType.DMA,
        ],
        compiler_params=pltpu.CompilerParams(disable_bounds_checks=True),
    )
    def k(table_hbm, idx_hbm, out_hbm, idx_v, rows_v, s0, s1, s2, s3):
        wid = lax.axis_index("s") * _NC + lax.axis_index("c")
        sems = (s0, s1, s2, s3)
        pltpu.sync_copy(idx_hbm.at[wid], idx_v)

        def fire(g, carry):
            tv = idx_v[g >> 3, pl.ds((g & 7) * 16, 16)]
            for l in range(16):
                pltpu.async_copy(
                    table_hbm.at[tv[l]],
                    rows_v.at[2 * g + (l // 8), l & 7],
                    sems[l % 4],
                )
            return carry

        lax.fori_loop(0, _B_PER_W // 16, fire, 0)

        for q in range(4):
            pltpu.make_async_copy(
                table_hbm.at[pl.ds(0, _B_PER_W // 32), :],
                rows_v.at[pl.ds(0, _B_PER_W // 32)],
                sems[q],
            ).wait()

        out3 = out_hbm.reshape(BATCH // 8, 8, EMBED_DIM)
        pltpu.sync_copy(rows_v, out3.at[pl.ds(wid * (_B_PER_W // 8),
                                              _B_PER_W // 8)])

    return k


_gather = _make_gather()


def kernel(x, table):
    idx3 = x.astype(jnp.int32).reshape(_NW, _NCHUNK, _CHUNK)
    return _gather(table, idx3)


# speedup vs baseline: 1.7304x; 1.0022x over previous
import functools

import jax
import jax.numpy as jnp
from jax import lax
from jax.experimental import pallas as pl
from jax.experimental.pallas import tpu as pltpu
from jax.experimental.pallas import tpu_sc as plsc

NUM_UNIQUE = 1000000
EMBED_DIM = 64
BATCH = 16384

_NC = 2
_NS = 16
_NW = _NC * _NS
_B_PER_W = BATCH // _NW
_CHUNK = 128
_NCHUNK = _B_PER_W // _CHUNK


def _make_gather():
    mesh = plsc.VectorSubcoreMesh(core_axis_name="c", subcore_axis_name="s")

    @functools.partial(
        pl.kernel,
        mesh=mesh,
        out_type=jax.ShapeDtypeStruct((BATCH, EMBED_DIM), jnp.float32),
        scratch_types=[
            pltpu.VMEM((4, 128), jnp.int32),
            pltpu.VMEM((_B_PER_W // 8, 8, EMBED_DIM), jnp.float32),
            pltpu.SemaphoreType.DMA,
            pltpu.SemaphoreType.DMA,
            pltpu.SemaphoreType.DMA,
            pltpu.SemaphoreType.DMA,
        ],
        compiler_params=pltpu.CompilerParams(disable_bounds_checks=True),
    )
    def k(table_hbm, idx_hbm, out_hbm, idx_v, rows_v, s0, s1, s2, s3):
        wid = lax.axis_index("s") * _NC + lax.axis_index("c")
        sems = (s0, s1, s2, s3)
        pltpu.sync_copy(idx_hbm.at[wid], idx_v)

        def fire(g, carry):
            tv = idx_v[g >> 3, pl.ds((g & 7) * 16, 16)]
            for l in range(16):
                pltpu.async_copy(
                    table_hbm.at[tv[l]],
                    rows_v.at[2 * g + (l // 8), l & 7],
                    sems[l % 4],
                )
            return carry

        plsc.parallel_loop(0, _B_PER_W // 16, unroll=4)(
            lambda g: fire(g, 0)
        )

        for q in range(4):
            pltpu.make_async_copy(
                table_hbm.at[pl.ds(0, _B_PER_W // 32), :],
                rows_v.at[pl.ds(0, _B_PER_W // 32)],
                sems[q],
            ).wait()

        out3 = out_hbm.reshape(BATCH // 8, 8, EMBED_DIM)
        pltpu.sync_copy(rows_v, out3.at[pl.ds(wid * (_B_PER_W // 8),
                                              _B_PER_W // 8)])

    return k


_gather = _make_gather()


def kernel(x, table):
    idx3 = x.astype(jnp.int32).reshape(_NW, _NCHUNK, _CHUNK)
    return _gather(table, idx3)
